# Initial kernel scaffold; baseline (speedup 1.0000x reference)
import jax
import jax.numpy as jnp
from jax.experimental import pallas as pl


def kernel(input_ids, wte, wpe):
    raise NotImplementedError("write your pallas kernel here")


# speedup vs baseline: 1.1264x; 1.1264x over previous
import functools

import jax
import jax.numpy as jnp
from jax import lax
from jax.experimental import pallas as pl
from jax.experimental.pallas import tpu as pltpu
from jax.experimental.pallas import tpu_sc as plsc

BATCH = 4
SEQ = 2048
D = 1024
NC = 2
NS = 16
NW = NC * NS
L = 16

POS_PER_W = SEQ // NW
CHUNK = 32
N_HALF = POS_PER_W // CHUNK
VECS = CHUNK * (D // L)

_mesh = plsc.VectorSubcoreMesh(core_axis_name="c", subcore_axis_name="s")


@functools.partial(
    pl.kernel,
    mesh=_mesh,
    out_type=jax.ShapeDtypeStruct((BATCH, SEQ, D), jnp.float32),
    scratch_types=[
        pltpu.VMEM((CHUNK,), jnp.int32),
        pltpu.VMEM((CHUNK, D), jnp.float32),
        pltpu.VMEM((CHUNK, D), jnp.float32),
        pltpu.SemaphoreType.DMA,
    ],
)
def _embed(ids_hbm, wte_hbm, wpe_hbm, out_hbm, idx_v, rows_v, wpe_v, sem):
    wid = lax.axis_index("s") * NC + lax.axis_index("c")
    p0 = wid * POS_PER_W
    for h in range(N_HALF):
        pos = p0 + h * CHUNK
        pltpu.sync_copy(wpe_hbm.at[pl.ds(pos, CHUNK)], wpe_v)
        for b in range(BATCH):
            pltpu.sync_copy(ids_hbm.at[b, pl.ds(pos, CHUNK)], idx_v)
            pltpu.async_copy(wte_hbm.at[idx_v], rows_v, sem).wait()

            def add_body(k, carry):
                i = k >> 6
                j = pl.multiple_of((k & 63) << 4, L)
                plsc.addupdate(rows_v.at[i, pl.ds(j, L)], wpe_v[i, pl.ds(j, L)])
                return carry

            lax.fori_loop(0, VECS, add_body, 0, unroll=4)
            pltpu.sync_copy(rows_v, out_hbm.at[b, pl.ds(pos, CHUNK)])


def kernel(input_ids, wte, wpe):
    out = _embed(input_ids.astype(jnp.int32), wte, wpe)
    return out


# speedup vs baseline: 1.4113x; 1.2529x over previous
import functools

import jax
import jax.numpy as jnp
from jax import lax
from jax.experimental import pallas as pl
from jax.experimental.pallas import tpu as pltpu
from jax.experimental.pallas import tpu_sc as plsc

BATCH = 4
SEQ = 2048
D = 1024
NC = 2
NS = 16
NW = NC * NS
L = 16

POS_PER_W = SEQ // NW
CHUNK = 32
N_HALF = POS_PER_W // CHUNK
NCHUNK = N_HALF * BATCH
VECS = CHUNK * (D // L)

_mesh = plsc.VectorSubcoreMesh(core_axis_name="c", subcore_axis_name="s")


@functools.partial(
    pl.kernel,
    mesh=_mesh,
    out_type=jax.ShapeDtypeStruct((BATCH, SEQ, D), jnp.float32),
    scratch_types=[
        pltpu.VMEM((BATCH, POS_PER_W), jnp.int32),
        pltpu.VMEM((CHUNK, D), jnp.float32),
        pltpu.VMEM((CHUNK, D), jnp.float32),
        pltpu.VMEM((CHUNK, D), jnp.float32),
        pltpu.SemaphoreType.DMA,
        pltpu.SemaphoreType.DMA,
    ],
)
def _embed(ids_hbm, wte_hbm, wpe_hbm, out_hbm, ids_v, rows_a, rows_b, wpe_v,
           sem_g, sem_s):
    wid = lax.axis_index("s") * NC + lax.axis_index("c")
    p0 = wid * POS_PER_W

    for b in range(BATCH):
        pltpu.sync_copy(ids_hbm.at[b, pl.ds(p0, POS_PER_W)], ids_v.at[b])

    rows = [rows_a, rows_b]

    def chunk_coords(t):
        h, b = divmod(t, BATCH)
        return h, b

    def start_gather(t):
        h, b = chunk_coords(t)
        return pltpu.async_copy(
            wte_hbm.at[ids_v.at[b, pl.ds(h * CHUNK, CHUNK)]],
            rows[t % 2], sem_g)

    def start_store(t):
        h, b = chunk_coords(t)
        return pltpu.async_copy(
            rows[t % 2], out_hbm.at[b, pl.ds(p0 + h * CHUNK, CHUNK)], sem_s)

    gathers = [None] * NCHUNK
    stores = [None] * NCHUNK

    gathers[0] = start_gather(0)
    for t in range(NCHUNK):
        if t + 1 < NCHUNK:
            if t - 1 >= 0:
                stores[t - 1].wait()
            gathers[t + 1] = start_gather(t + 1)
        gathers[t].wait()
        h, b = chunk_coords(t)
        if b == 0:
            pltpu.sync_copy(wpe_hbm.at[pl.ds(p0 + h * CHUNK, CHUNK)], wpe_v)
        buf = rows[t % 2]

        def add_body(k, carry):
            i = k >> 6
            j = pl.multiple_of((k & 63) << 4, L)
            plsc.addupdate(buf.at[i, pl.ds(j, L)], wpe_v[i, pl.ds(j, L)])
            return carry

        lax.fori_loop(0, VECS, add_body, 0, unroll=4)
        stores[t] = start_store(t)
    stores[NCHUNK - 2].wait()
    stores[NCHUNK - 1].wait()


def kernel(input_ids, wte, wpe):
    return _embed(input_ids.astype(jnp.int32), wte, wpe)
